# baseline (speedup 1.0000x reference)
import functools

import jax
import jax.numpy as jnp
from jax import lax
from jax.experimental import pallas as pl
from jax.experimental.pallas import tpu as pltpu
from jax.experimental.pallas import tpu_sc as plsc

N = 10000
E = 320000
F = 128
NG = 100
NGP = 128
NP = 10240
NTILES = 32
EPT = NP
EP = NTILES * EPT
CH = 128
NCH = EPT // CH
RPS = NP // 16
RB = 1024
EB = 1024

def _sc_counts_body(src_hbm, dst_hbm, out_hbm, ebs, ebd, cs, cd):
    c = lax.axis_index("c")
    s = lax.axis_index("s")
    w = c * 16 + s
    rbase = w * NCH
    erows = EB // CH

    def zfill(j, _):
        cs[pl.ds(j * 16, 16)] = jnp.zeros((16,), jnp.float32)
        cd[pl.ds(j * 16, 16)] = jnp.zeros((16,), jnp.float32)
        return 0

    lax.fori_loop(0, NP // 16, zfill, 0)

    ones = jnp.ones((16,), jnp.float32)

    def chunk(g, _):
        pltpu.sync_copy(src_hbm.at[pl.ds(rbase + g * erows, erows)], ebs)
        pltpu.sync_copy(dst_hbm.at[pl.ds(rbase + g * erows, erows)], ebd)

        def inner(r, _):
            def lane(l, _):
                plsc.addupdate_scatter(cs, [ebs[r, pl.ds(l * 16, 16)]], ones)
                plsc.addupdate_scatter(cd, [ebd[r, pl.ds(l * 16, 16)]], ones)
                return 0

            lax.fori_loop(0, CH // 16, lane, 0)
            return 0

        lax.fori_loop(0, erows, inner, 0)
        return 0

    lax.fori_loop(0, NCH // erows, chunk, 0)

    pltpu.sync_copy(cs, out_hbm.at[pl.ds(w * 2 * NP, NP)])
    pltpu.sync_copy(cd, out_hbm.at[pl.ds(w * 2 * NP + NP, NP)])


def _sc_edge_pass_body(src_hbm, dst_hbm, table_hbm, out_hbm, idx_s, idx_d,
                       rows, gsem, acc):
    c = lax.axis_index("c")
    s = lax.axis_index("s")
    rbase = (c * 16 + s) * NCH

    def zrow(j, _):
        def zlane(l, _):
            rows[j, pl.ds(l * 16, 16)] = jnp.zeros((16,), jnp.float32)
            return 0

        lax.fori_loop(0, F // 16, zlane, 0)
        return 0

    lax.fori_loop(0, CH, zrow, 0)

    def zcopy(k, _):
        pltpu.sync_copy(rows, acc.at[pl.ds(s * RPS + k * CH, CH)])
        return 0

    lax.fori_loop(0, RPS // CH, zcopy, 0)
    plsc.subcore_barrier()

    def chunk(g, _):
        r = rbase + g
        pltpu.sync_copy(src_hbm.at[r], idx_s)
        pltpu.sync_copy(dst_hbm.at[r], idx_d)
        pltpu.async_copy(table_hbm.at[idx_s], rows, gsem).wait()
        pltpu.sync_copy(rows, acc.at[idx_d], add=True)
        return 0

    lax.fori_loop(0, NCH, chunk, 0)
    plsc.subcore_barrier()

    pltpu.sync_copy(acc.at[pl.ds(s * RPS, RPS)],
                    out_hbm.at[c, pl.ds(s * RPS, RPS)])


@functools.lru_cache(maxsize=None)
def _sc_kernels():
    mesh = plsc.VectorSubcoreMesh(core_axis_name="c", subcore_axis_name="s")
    counts = pl.kernel(
        _sc_counts_body,
        out_type=jax.ShapeDtypeStruct((NTILES * 2 * NP,), jnp.float32),
        mesh=mesh,
        scratch_types=[
            pltpu.VMEM((EB // CH, CH), jnp.int32),
            pltpu.VMEM((EB // CH, CH), jnp.int32),
            pltpu.VMEM((NP,), jnp.float32),
            pltpu.VMEM((NP,), jnp.float32),
        ],
        compiler_params=pltpu.CompilerParams(needs_layout_passes=False),
    )
    edge_pass = pl.kernel(
        _sc_edge_pass_body,
        out_type=jax.ShapeDtypeStruct((2, NP, F), jnp.float32),
        mesh=mesh,
        scratch_types=[
            pltpu.VMEM((CH,), jnp.int32),
            pltpu.VMEM((CH,), jnp.int32),
            pltpu.VMEM((CH, F), jnp.float32),
            pltpu.SemaphoreType.DMA,
            pltpu.VMEM_SHARED((NP, F), jnp.float32),
        ],
    )
    return counts, edge_pass


def _tc_prep_body(cnt_ref, x_ref, y_ref, nrm_ref):
    cnt = cnt_ref[...]
    cs = jnp.sum(cnt[:, :NTILES], axis=1, keepdims=True)
    cd = jnp.sum(cnt[:, NTILES:], axis=1, keepdims=True)
    ns = lax.rsqrt(jnp.maximum(cs, 1.0))
    nd = lax.rsqrt(jnp.maximum(cd, 1.0))
    y_ref[...] = x_ref[...] * ns
    nrm_ref[...] = jnp.concatenate([ns, nd], axis=1)


def _tc_mid_body(pp_ref, nrm_ref, w1_ref, b1_ref, h1_ref, ys_ref, ps_ref):
    P = pp_ref[0] + pp_ref[1]
    Z = jnp.dot(P, w1_ref[...], preferred_element_type=jnp.float32)
    ns = nrm_ref[:, 0:1]
    nd = nrm_ref[:, 1:2]
    H1 = jnp.maximum(Z * nd + b1_ref[...], 0.0)
    h1_ref[...] = H1
    ys_ref[...] = H1 * ns
    ps_ref[...] = P


def _tc_out_body(qp_ref, ps_ref, h1_ref, nrm_ref, g_ref, w2a_ref, w2b_ref,
                 b2_ref, hg_ref, sums, cnts):
    i = pl.program_id(0)
    Q = qp_ref[0] + qp_ref[1]
    Z = (jnp.dot(Q, w2a_ref[...], preferred_element_type=jnp.float32)
         + jnp.dot(ps_ref[...], w2b_ref[...], preferred_element_type=jnp.float32))
    nd = nrm_ref[:, 1:2]
    H2 = jnp.maximum(Z * nd + b2_ref[...], 0.0)
    feat = jnp.concatenate([h1_ref[...], H2], axis=1)
    gio = lax.broadcasted_iota(jnp.int32, (RB, NGP), 1)
    onehot = (g_ref[...] == gio).astype(jnp.float32)
    contrib = lax.dot_general(onehot, feat, (((0,), (0,)), ((), ())),
                              preferred_element_type=jnp.float32)
    ccnt = lax.dot_general(onehot, jnp.ones((RB, 1), jnp.float32),
                           (((0,), (0,)), ((), ())),
                           preferred_element_type=jnp.float32)

    @pl.when(i == 0)
    def _():
        sums[...] = jnp.zeros_like(sums)
        cnts[...] = jnp.zeros_like(cnts)

    sums[...] += contrib
    cnts[...] += ccnt

    @pl.when(i == pl.num_programs(0) - 1)
    def _():
        hg_ref[...] = sums[...] / jnp.maximum(cnts[...], 1.0)


_tc_prep = pl.pallas_call(
    _tc_prep_body,
    grid=(NP // RB,),
    in_specs=[
        pl.BlockSpec((RB, 2 * NTILES), lambda i: (i, 0)),
        pl.BlockSpec((RB, F), lambda i: (i, 0)),
    ],
    out_specs=[
        pl.BlockSpec((RB, F), lambda i: (i, 0)),
        pl.BlockSpec((RB, 2), lambda i: (i, 0)),
    ],
    out_shape=[
        jax.ShapeDtypeStruct((NP, F), jnp.float32),
        jax.ShapeDtypeStruct((NP, 2), jnp.float32),
    ],
)

_tc_mid = pl.pallas_call(
    _tc_mid_body,
    grid=(NP // RB,),
    in_specs=[
        pl.BlockSpec((2, RB, F), lambda i: (0, i, 0)),
        pl.BlockSpec((RB, 2), lambda i: (i, 0)),
        pl.BlockSpec((F, F), lambda i: (0, 0)),
        pl.BlockSpec((1, F), lambda i: (0, 0)),
    ],
    out_specs=[
        pl.BlockSpec((RB, F), lambda i: (i, 0)),
        pl.BlockSpec((RB, F), lambda i: (i, 0)),
        pl.BlockSpec((RB, F), lambda i: (i, 0)),
    ],
    out_shape=[
        jax.ShapeDtypeStruct((NP, F), jnp.float32),
        jax.ShapeDtypeStruct((NP, F), jnp.float32),
        jax.ShapeDtypeStruct((NP, F), jnp.float32),
    ],
)

_tc_out = pl.pallas_call(
    _tc_out_body,
    grid=(NP // RB,),
    in_specs=[
        pl.BlockSpec((2, RB, F), lambda i: (0, i, 0)),
        pl.BlockSpec((RB, F), lambda i: (i, 0)),
        pl.BlockSpec((RB, F), lambda i: (i, 0)),
        pl.BlockSpec((RB, 2), lambda i: (i, 0)),
        pl.BlockSpec((RB, 1), lambda i: (i, 0)),
        pl.BlockSpec((F, F), lambda i: (0, 0)),
        pl.BlockSpec((F, F), lambda i: (0, 0)),
        pl.BlockSpec((1, F), lambda i: (0, 0)),
    ],
    out_specs=pl.BlockSpec((NGP, 2 * F), lambda i: (0, 0)),
    out_shape=jax.ShapeDtypeStruct((NGP, 2 * F), jnp.float32),
    scratch_shapes=[
        pltpu.VMEM((NGP, 2 * F), jnp.float32),
        pltpu.VMEM((NGP, 1), jnp.float32),
    ],
)


def kernel(x, edge_index, graph_ids, W1, b1, W2, b2):
    src = edge_index[0]
    dst = edge_index[1]
    pad_e = EP - E
    epad = jnp.full((pad_e,), N, jnp.int32)
    srcp = jnp.concatenate([src, epad]).reshape(EP // CH, CH)
    dstp = jnp.concatenate([dst, epad]).reshape(EP // CH, CH)
    xp = jnp.pad(x, ((0, NP - N), (0, 0)))
    gcol = jnp.concatenate(
        [graph_ids, jnp.full((NP - N,), NGP - 1, jnp.int32)]).reshape(NP, 1)

    sc_counts, sc_edge_pass = _sc_kernels()
    cntf = sc_counts(srcp, dstp)
    cntT = cntf.reshape(NTILES, 2, NP).transpose(2, 1, 0).reshape(NP, 2 * NTILES)
    y, nrm = _tc_prep(cntT, xp)
    pp = sc_edge_pass(srcp, dstp, y)
    h1, ys, ps = _tc_mid(pp, nrm, W1, b1.reshape(1, F))
    qp = sc_edge_pass(srcp, dstp, ys)
    hgp = _tc_out(qp, ps, h1, nrm, gcol, W2[:F], W2[F:], b2.reshape(1, F))
    return hgp[:NG]


# speedup vs baseline: 1.1693x; 1.1693x over previous
import functools

import jax
import jax.numpy as jnp
from jax import lax
from jax.experimental import pallas as pl
from jax.experimental.pallas import tpu as pltpu
from jax.experimental.pallas import tpu_sc as plsc

N = 10000
E = 320000
F = 128
NG = 100
NGP = 128
NP = 10240
NTILES = 32
EPT = NP
EP = NTILES * EPT
CH = 128
NCH = EPT // CH
RPS = NP // 16
RB = 1024
EB = 1024

def _sc_counts_body(src_hbm, dst_hbm, out_hbm, ebs, ebd, cs, cd):
    c = lax.axis_index("c")
    s = lax.axis_index("s")
    w = c * 16 + s

    def zfill(j, _):
        cs[pl.ds(j * 16, 16)] = jnp.zeros((16,), jnp.float32)
        cd[pl.ds(j * 16, 16)] = jnp.zeros((16,), jnp.float32)
        return 0

    lax.fori_loop(0, NP // 16, zfill, 0)

    ones = jnp.ones((16,), jnp.float32)
    ebase = w * EPT

    def chunk(g, _):
        pltpu.sync_copy(src_hbm.at[pl.ds(ebase + g * EB, EB)], ebs)
        pltpu.sync_copy(dst_hbm.at[pl.ds(ebase + g * EB, EB)], ebd)

        def inner(j, _):
            plsc.addupdate_scatter(cs, [ebs[pl.ds(j * 16, 16)]], ones)
            plsc.addupdate_scatter(cd, [ebd[pl.ds(j * 16, 16)]], ones)
            return 0

        lax.fori_loop(0, EB // 16, inner, 0)
        return 0

    lax.fori_loop(0, EPT // EB, chunk, 0)

    pltpu.sync_copy(cs, out_hbm.at[pl.ds(w * 2 * NP, NP)])
    pltpu.sync_copy(cd, out_hbm.at[pl.ds(w * 2 * NP + NP, NP)])


def _sc_edge_pass_body(src_hbm, dst_hbm, table_hbm, out_hbm, idx_s, idx_d,
                       rows, gsem, acc):
    c = lax.axis_index("c")
    s = lax.axis_index("s")
    ebase = (c * 16 + s) * EPT

    def zrow(j, _):
        def zlane(l, _):
            rows[j, pl.ds(l * 16, 16)] = jnp.zeros((16,), jnp.float32)
            return 0

        lax.fori_loop(0, F // 16, zlane, 0)
        return 0

    lax.fori_loop(0, CH, zrow, 0)

    def zcopy(k, _):
        pltpu.sync_copy(rows, acc.at[pl.ds(s * RPS + k * CH, CH)])
        return 0

    lax.fori_loop(0, RPS // CH, zcopy, 0)
    plsc.subcore_barrier()

    def chunk(g, _):
        off = ebase + g * CH
        pltpu.sync_copy(src_hbm.at[pl.ds(off, CH)], idx_s)
        pltpu.sync_copy(dst_hbm.at[pl.ds(off, CH)], idx_d)
        pltpu.async_copy(table_hbm.at[idx_s], rows, gsem).wait()
        pltpu.sync_copy(rows, acc.at[idx_d], add=True)
        return 0

    lax.fori_loop(0, NCH, chunk, 0)
    plsc.subcore_barrier()

    pltpu.sync_copy(acc.at[pl.ds(s * RPS, RPS)],
                    out_hbm.at[c, pl.ds(s * RPS, RPS)])


@functools.lru_cache(maxsize=None)
def _sc_kernels():
    mesh = plsc.VectorSubcoreMesh(core_axis_name="c", subcore_axis_name="s")
    counts = pl.kernel(
        _sc_counts_body,
        out_type=jax.ShapeDtypeStruct((NTILES * 2 * NP,), jnp.float32),
        mesh=mesh,
        scratch_types=[
            pltpu.VMEM((EB,), jnp.int32),
            pltpu.VMEM((EB,), jnp.int32),
            pltpu.VMEM((NP,), jnp.float32),
            pltpu.VMEM((NP,), jnp.float32),
        ],
        compiler_params=pltpu.CompilerParams(needs_layout_passes=False),
    )
    edge_pass = pl.kernel(
        _sc_edge_pass_body,
        out_type=jax.ShapeDtypeStruct((2, NP, F), jnp.float32),
        mesh=mesh,
        scratch_types=[
            pltpu.VMEM((CH,), jnp.int32),
            pltpu.VMEM((CH,), jnp.int32),
            pltpu.VMEM((CH, F), jnp.float32),
            pltpu.SemaphoreType.DMA,
            pltpu.VMEM_SHARED((NP, F), jnp.float32),
        ],
    )
    return counts, edge_pass


def _tc_prep_body(cnt_ref, x_ref, y_ref, nrm_ref):
    cnt = cnt_ref[...]
    cs = jnp.sum(cnt[:, :NTILES], axis=1, keepdims=True)
    cd = jnp.sum(cnt[:, NTILES:], axis=1, keepdims=True)
    ns = lax.rsqrt(jnp.maximum(cs, 1.0))
    nd = lax.rsqrt(jnp.maximum(cd, 1.0))
    y_ref[...] = x_ref[...] * ns
    nrm_ref[...] = jnp.concatenate([ns, nd], axis=1)


def _tc_mid_body(pp_ref, nrm_ref, w1_ref, b1_ref, h1_ref, ys_ref, ps_ref):
    P = pp_ref[0] + pp_ref[1]
    Z = jnp.dot(P, w1_ref[...], preferred_element_type=jnp.float32)
    ns = nrm_ref[:, 0:1]
    nd = nrm_ref[:, 1:2]
    H1 = jnp.maximum(Z * nd + b1_ref[...], 0.0)
    h1_ref[...] = H1
    ys_ref[...] = H1 * ns
    ps_ref[...] = P


def _tc_out_body(qp_ref, ps_ref, h1_ref, nrm_ref, g_ref, w2a_ref, w2b_ref,
                 b2_ref, hg_ref, sums, cnts):
    i = pl.program_id(0)
    Q = qp_ref[0] + qp_ref[1]
    Z = (jnp.dot(Q, w2a_ref[...], preferred_element_type=jnp.float32)
         + jnp.dot(ps_ref[...], w2b_ref[...], preferred_element_type=jnp.float32))
    nd = nrm_ref[:, 1:2]
    H2 = jnp.maximum(Z * nd + b2_ref[...], 0.0)
    feat = jnp.concatenate([h1_ref[...], H2], axis=1)
    gio = lax.broadcasted_iota(jnp.int32, (RB, NGP), 1)
    onehot = (g_ref[...] == gio).astype(jnp.float32)
    contrib = lax.dot_general(onehot, feat, (((0,), (0,)), ((), ())),
                              preferred_element_type=jnp.float32)
    ccnt = lax.dot_general(onehot, jnp.ones((RB, 1), jnp.float32),
                           (((0,), (0,)), ((), ())),
                           preferred_element_type=jnp.float32)

    @pl.when(i == 0)
    def _():
        sums[...] = jnp.zeros_like(sums)
        cnts[...] = jnp.zeros_like(cnts)

    sums[...] += contrib
    cnts[...] += ccnt

    @pl.when(i == pl.num_programs(0) - 1)
    def _():
        hg_ref[...] = sums[...] / jnp.maximum(cnts[...], 1.0)


_tc_prep = pl.pallas_call(
    _tc_prep_body,
    grid=(NP // RB,),
    in_specs=[
        pl.BlockSpec((RB, 2 * NTILES), lambda i: (i, 0)),
        pl.BlockSpec((RB, F), lambda i: (i, 0)),
    ],
    out_specs=[
        pl.BlockSpec((RB, F), lambda i: (i, 0)),
        pl.BlockSpec((RB, 2), lambda i: (i, 0)),
    ],
    out_shape=[
        jax.ShapeDtypeStruct((NP, F), jnp.float32),
        jax.ShapeDtypeStruct((NP, 2), jnp.float32),
    ],
)

_tc_mid = pl.pallas_call(
    _tc_mid_body,
    grid=(NP // RB,),
    in_specs=[
        pl.BlockSpec((2, RB, F), lambda i: (0, i, 0)),
        pl.BlockSpec((RB, 2), lambda i: (i, 0)),
        pl.BlockSpec((F, F), lambda i: (0, 0)),
        pl.BlockSpec((1, F), lambda i: (0, 0)),
    ],
    out_specs=[
        pl.BlockSpec((RB, F), lambda i: (i, 0)),
        pl.BlockSpec((RB, F), lambda i: (i, 0)),
        pl.BlockSpec((RB, F), lambda i: (i, 0)),
    ],
    out_shape=[
        jax.ShapeDtypeStruct((NP, F), jnp.float32),
        jax.ShapeDtypeStruct((NP, F), jnp.float32),
        jax.ShapeDtypeStruct((NP, F), jnp.float32),
    ],
)

_tc_out = pl.pallas_call(
    _tc_out_body,
    grid=(NP // RB,),
    in_specs=[
        pl.BlockSpec((2, RB, F), lambda i: (0, i, 0)),
        pl.BlockSpec((RB, F), lambda i: (i, 0)),
        pl.BlockSpec((RB, F), lambda i: (i, 0)),
        pl.BlockSpec((RB, 2), lambda i: (i, 0)),
        pl.BlockSpec((RB, 1), lambda i: (i, 0)),
        pl.BlockSpec((F, F), lambda i: (0, 0)),
        pl.BlockSpec((F, F), lambda i: (0, 0)),
        pl.BlockSpec((1, F), lambda i: (0, 0)),
    ],
    out_specs=pl.BlockSpec((NGP, 2 * F), lambda i: (0, 0)),
    out_shape=jax.ShapeDtypeStruct((NGP, 2 * F), jnp.float32),
    scratch_shapes=[
        pltpu.VMEM((NGP, 2 * F), jnp.float32),
        pltpu.VMEM((NGP, 1), jnp.float32),
    ],
)


def kernel(x, edge_index, graph_ids, W1, b1, W2, b2):
    src = edge_index[0]
    dst = edge_index[1]
    pad_e = EP - E
    epad = jnp.full((pad_e,), N, jnp.int32)
    srcp = jnp.concatenate([src, epad])
    dstp = jnp.concatenate([dst, epad])
    xp = jnp.pad(x, ((0, NP - N), (0, 0)))
    gcol = jnp.concatenate(
        [graph_ids, jnp.full((NP - N,), NGP - 1, jnp.int32)]).reshape(NP, 1)

    sc_counts, sc_edge_pass = _sc_kernels()
    cntf = sc_counts(srcp, dstp)
    cntT = cntf.reshape(NTILES, 2, NP).transpose(2, 1, 0).reshape(NP, 2 * NTILES)
    y, nrm = _tc_prep(cntT, xp)
    pp = sc_edge_pass(srcp, dstp, y)
    h1, ys, ps = _tc_mid(pp, nrm, W1, b1.reshape(1, F))
    qp = sc_edge_pass(srcp, dstp, ys)
    hgp = _tc_out(qp, ps, h1, nrm, gcol, W2[:F], W2[F:], b2.reshape(1, F))
    return hgp[:NG]
